# baseline (speedup 1.0000x reference)
import jax
import jax.numpy as jnp
from jax.experimental import pallas as pl
from jax.experimental.pallas import tpu as pltpu

_PP = 2
_NBUF = 6
_NXS = 4


def _rnn_kernel(s_ref, x_hbm, o_ref, buf_ref, xs_ref, sem_ref):
    j = pl.program_id(0)
    n_p = (pl.num_programs(0) - 1) * _PP
    a = s_ref[0]
    b = s_ref[1]
    c = s_ref[2]
    bl = xs_ref.shape[3]
    ids = a * jnp.eye(128, dtype=jnp.float32)

    @pl.when(j == 0)
    def _prologue():
        o_ref[...] = jnp.zeros_like(o_ref)
        for d in range(_NBUF):
            pltpu.make_async_copy(
                x_hbm.at[:, d, :], buf_ref.at[d], sem_ref.at[d]
            ).start()

    @pl.when(j < n_p // _PP)
    def _relayout():
        for q in range(_PP):
            p = j * _PP + q
            slot = p % _NBUF
            pltpu.make_async_copy(
                x_hbm.at[:, p, :], buf_ref.at[slot], sem_ref.at[slot]
            ).wait()
            w = buf_ref[slot].reshape(8, bl, 128)
            xs_ref[p % _NXS] = jax.lax.dot_general(
                ids, w, (((1,), (2,)), ((), ())),
                preferred_element_type=jnp.float32,
            )
            nxt = p + _NBUF

            @pl.when(nxt < n_p)
            def _prefetch():
                pltpu.make_async_copy(
                    x_hbm.at[:, nxt, :], buf_ref.at[slot], sem_ref.at[slot]
                ).start()

    @pl.when(j > 0)
    def _recurrence():
        h = o_ref[...]
        for q in range(_PP):
            rslot = ((j - 1) * _PP + q) % _NXS
            for k in range(128):
                u = xs_ref[rslot, k] + c
                h = jnp.tanh(h * b + u)
        o_ref[...] = h


def kernel(x, w_ih, w_hh, b_ih, b_hh):
    B, T, _ = x.shape
    BL = B // 8
    NP = T // 128

    xv = x.reshape(B, T // 128, 128)
    scal = jnp.stack([w_ih[0, 0], w_hh[0, 0], b_ih[0] + b_hh[0]])

    out = pl.pallas_call(
        _rnn_kernel,
        grid=(NP // _PP + 1,),
        in_specs=[
            pl.BlockSpec(memory_space=pltpu.SMEM),
            pl.BlockSpec(memory_space=pl.ANY),
        ],
        out_specs=pl.BlockSpec((8, BL), lambda j: (0, 0)),
        out_shape=jax.ShapeDtypeStruct((8, BL), x.dtype),
        scratch_shapes=[
            pltpu.VMEM((_NBUF, B, 128), x.dtype),
            pltpu.VMEM((_NXS, 128, 8, BL), x.dtype),
            pltpu.SemaphoreType.DMA((_NBUF,)),
        ],
        compiler_params=pltpu.CompilerParams(
            dimension_semantics=("arbitrary",),
            vmem_limit_bytes=56 * 1024 * 1024,
        ),
    )(scal, xv)

    return out.reshape(B, 1)


# speedup vs baseline: 1.0028x; 1.0028x over previous
import jax
import jax.numpy as jnp
from jax.experimental import pallas as pl
from jax.experimental.pallas import tpu as pltpu

_PP = 2
_NBUF = 6
_NXS = 4


def _rnn_kernel(s_ref, x_hbm, o_ref, buf_ref, xs_ref, sem_ref):
    j = pl.program_id(0)
    n_p = (pl.num_programs(0) - 1) * _PP
    a = s_ref[0]
    b = s_ref[1]
    c = s_ref[2]
    bl = xs_ref.shape[3]

    @pl.when(j == 0)
    def _prologue():
        o_ref[...] = jnp.zeros_like(o_ref)
        for d in range(_NBUF):
            pltpu.make_async_copy(
                x_hbm.at[:, d, :], buf_ref.at[d], sem_ref.at[d]
            ).start()

    @pl.when(j < n_p // _PP)
    def _relayout():
        for q in range(_PP):
            p = j * _PP + q
            slot = p % _NBUF
            pltpu.make_async_copy(
                x_hbm.at[:, p, :], buf_ref.at[slot], sem_ref.at[slot]
            ).wait()
            v = buf_ref[slot].reshape(8, bl, 128)
            xs_ref[p % _NXS] = jnp.transpose(v, (2, 0, 1))
            nxt = p + _NBUF

            @pl.when(nxt < n_p)
            def _prefetch():
                pltpu.make_async_copy(
                    x_hbm.at[:, nxt, :], buf_ref.at[slot], sem_ref.at[slot]
                ).start()

    @pl.when(j > 0)
    def _recurrence():
        h = o_ref[...]
        for q in range(_PP):
            rslot = ((j - 1) * _PP + q) % _NXS
            for k in range(128):
                u = xs_ref[rslot, k] * a + c
                h = jnp.tanh(h * b + u)
        o_ref[...] = h


def kernel(x, w_ih, w_hh, b_ih, b_hh):
    B, T, _ = x.shape
    BL = B // 8
    NP = T // 128

    xv = x.reshape(B, T // 128, 128)
    scal = jnp.stack([w_ih[0, 0], w_hh[0, 0], b_ih[0] + b_hh[0]])

    out = pl.pallas_call(
        _rnn_kernel,
        grid=(NP // _PP + 1,),
        in_specs=[
            pl.BlockSpec(memory_space=pltpu.SMEM),
            pl.BlockSpec(memory_space=pl.ANY),
        ],
        out_specs=pl.BlockSpec((8, BL), lambda j: (0, 0)),
        out_shape=jax.ShapeDtypeStruct((8, BL), x.dtype),
        scratch_shapes=[
            pltpu.VMEM((_NBUF, B, 128), x.dtype),
            pltpu.VMEM((_NXS, 128, 8, BL), x.dtype),
            pltpu.SemaphoreType.DMA((_NBUF,)),
        ],
        compiler_params=pltpu.CompilerParams(
            dimension_semantics=("arbitrary",),
            vmem_limit_bytes=56 * 1024 * 1024,
        ),
    )(scal, xv)

    return out.reshape(B, 1)
